# baseline (speedup 1.0000x reference)
import functools

import jax
import jax.numpy as jnp
from jax import lax
from jax.experimental import pallas as pl
from jax.experimental.pallas import tpu as pltpu
from jax.experimental.pallas import tpu_sc as plsc

N = 10000
D = 256
HALF = 128
E = 160000

NC = 2
NS = 16
N_PAD = 10240
ROWS_PER_TILE = N_PAD // NS
EDGES_PER_TILE = E // NS
EPT_PAD = 10080
AGG_CHUNK = 72
AGG_ITERS = EPT_PAD // AGG_CHUNK
DEG_EDGES_PER_TILE = E // (NC * NS)
DEG_EPT_PAD = 5200
DEG_CHUNK = 104
DEG_ITERS = DEG_EPT_PAD // DEG_CHUNK

BR = 400
GRID = N // BR


def _mesh():
    return plsc.VectorSubcoreMesh(core_axis_name="c", subcore_axis_name="s")


def _sc_degree(dst_deg, ones_hbm, zeros_hbm):
    @functools.partial(
        pl.kernel,
        out_type=jax.ShapeDtypeStruct((NC, N_PAD, HALF), jnp.float32),
        mesh=_mesh(),
        scratch_types=[
            pltpu.VMEM((DEG_EPT_PAD,), jnp.int32),
            pltpu.VMEM((DEG_CHUNK, HALF), jnp.float32),
            pltpu.VMEM_SHARED((N_PAD, HALF), jnp.float32),
            pltpu.SemaphoreType.DMA,
        ],
    )
    def k(dst_r, ones_r, zeros_r, d_r, didx, ones_v, acc, sem):
        c = lax.axis_index("c")
        s = lax.axis_index("s")
        row0 = s * ROWS_PER_TILE
        pltpu.sync_copy(zeros_r, acc.at[pl.ds(row0, ROWS_PER_TILE)])
        pltpu.sync_copy(ones_r, ones_v)
        pltpu.sync_copy(
            dst_r.at[pl.ds((c * NS + s) * DEG_EPT_PAD, DEG_EPT_PAD)], didx)
        plsc.subcore_barrier()

        def body(i, carry):
            pltpu.async_copy(
                ones_v, acc.at[didx.at[pl.ds(i * DEG_CHUNK, DEG_CHUNK)]],
                sem, add=True)
            return carry

        lax.fori_loop(0, DEG_ITERS, body, 0)

        def drain(i, carry):
            pltpu.make_async_copy(ones_r, ones_v, sem).wait()
            return carry

        lax.fori_loop(0, DEG_ITERS, drain, 0)
        plsc.subcore_barrier()
        sl = pl.ds(row0, ROWS_PER_TILE)
        pltpu.sync_copy(acc.at[sl], d_r.at[c, sl])

    return k(dst_deg, ones_hbm, zeros_hbm)


def _sl(idx_ref, i):
    return idx_ref.at[pl.ds(i * AGG_CHUNK, AGG_CHUNK)]


def _sc_aggregate(src2, dst2, y, zeros_hbm):
    @functools.partial(
        pl.kernel,
        out_type=jax.ShapeDtypeStruct((NC, N_PAD, HALF), jnp.float32),
        mesh=_mesh(),
        scratch_types=[
            pltpu.VMEM((NS * AGG_ITERS * AGG_CHUNK // NS,), jnp.int32),
            pltpu.VMEM((NS * AGG_ITERS * AGG_CHUNK // NS,), jnp.int32),
            pltpu.VMEM((AGG_CHUNK, HALF), jnp.float32),
            pltpu.VMEM((AGG_CHUNK, HALF), jnp.float32),
            pltpu.VMEM((AGG_CHUNK, HALF), jnp.float32),
            pltpu.VMEM_SHARED((N_PAD, HALF), jnp.float32),
            pltpu.SemaphoreType.DMA,
            pltpu.SemaphoreType.DMA,
            pltpu.SemaphoreType.DMA,
        ],
    )
    def k(src_r, dst_r, y_r, z_r, o_r, sidx, didx, rows0, rows1, rows2, acc,
          sem0, sem1, sem2):
        c = lax.axis_index("c")
        s = lax.axis_index("s")
        row0 = s * ROWS_PER_TILE
        pltpu.sync_copy(src_r.at[pl.ds((c * NS + s) * EPT_PAD, EPT_PAD)], sidx)
        pltpu.sync_copy(dst_r.at[pl.ds(s * EPT_PAD, EPT_PAD)], didx)
        pltpu.sync_copy(z_r, acc.at[pl.ds(row0, ROWS_PER_TILE)])
        pltpu.async_copy(y_r.at[_sl(sidx, 0)], rows0, sem0)
        pltpu.async_copy(y_r.at[_sl(sidx, 1)], rows1, sem1)
        plsc.subcore_barrier()

        def wait(rows, sem):
            pltpu.make_async_copy(y_r.at[_sl(sidx, 0)], rows, sem).wait()

        def body(i, carry):
            i0 = 3 * i
            wait(rows0, sem0)
            pltpu.async_copy(y_r.at[_sl(sidx, i0 + 2)], rows2, sem2)
            pltpu.sync_copy(rows0, acc.at[_sl(didx, i0)], add=True)
            wait(rows1, sem1)
            pltpu.async_copy(y_r.at[_sl(sidx, i0 + 3)], rows0, sem0)
            pltpu.sync_copy(rows1, acc.at[_sl(didx, i0 + 1)], add=True)
            wait(rows2, sem2)
            pltpu.async_copy(y_r.at[_sl(sidx, i0 + 4)], rows1, sem1)
            pltpu.sync_copy(rows2, acc.at[_sl(didx, i0 + 2)], add=True)
            return carry

        lax.fori_loop(0, (AGG_ITERS - 2) // 3, body, 0)
        wait(rows0, sem0)
        pltpu.sync_copy(rows0, acc.at[_sl(didx, AGG_ITERS - 2)], add=True)
        wait(rows1, sem1)
        pltpu.sync_copy(rows1, acc.at[_sl(didx, AGG_ITERS - 1)], add=True)
        plsc.subcore_barrier()
        sl = pl.ds(row0, ROWS_PER_TILE)
        pltpu.sync_copy(acc.at[sl], o_r.at[c, sl])

    return k(src2, dst2, y, zeros_hbm)


def _dis_block(d_r):
    d = d_r[0, :, 0:1] + d_r[1, :, 0:1]
    return jnp.where(d > 0, lax.rsqrt(d), 0.0)


_DEG_SPEC = pl.BlockSpec((NC, BR, HALF), lambda i: (0, i, 0))


def _tc_matmul1(x, W1):
    def body(x_r, w_r, y_r):
        y_r[...] = jnp.dot(x_r[...], w_r[...],
                           preferred_element_type=jnp.float32)

    return pl.pallas_call(
        body,
        grid=(GRID,),
        in_specs=[
            pl.BlockSpec((BR, D), lambda i: (i, 0)),
            pl.BlockSpec((D, D), lambda i: (0, 0)),
        ],
        out_specs=pl.BlockSpec((BR, D), lambda i: (i, 0)),
        out_shape=jax.ShapeDtypeStruct((N, D), jnp.float32),
    )(x, W1)


def _tc_scale1(xw, deg):
    def body(xw_r, d_r, y_r):
        dis = _dis_block(d_r)
        y = xw_r[...] * dis
        y_r[0] = y[:, :HALF]
        y_r[1] = y[:, HALF:]

    return pl.pallas_call(
        body,
        grid=(GRID,),
        in_specs=[
            pl.BlockSpec((BR, D), lambda i: (i, 0)),
            _DEG_SPEC,
        ],
        out_specs=pl.BlockSpec((NC, BR, HALF), lambda i: (0, i, 0)),
        out_shape=jax.ShapeDtypeStruct((NC, N, HALF), jnp.float32),
    )(xw, deg)


def _tc_layer2(agg, deg, b1, W2):
    def body(a_r, d_r, b_r, w_r, y_r):
        dis = _dis_block(d_r)
        ag = jnp.concatenate([a_r[0], a_r[1]], axis=1)
        h = jnp.maximum(ag * dis + b_r[...], 0.0)
        y = jnp.dot(h, w_r[...], preferred_element_type=jnp.float32) * dis
        y_r[0] = y[:, :HALF]
        y_r[1] = y[:, HALF:]

    return pl.pallas_call(
        body,
        grid=(GRID,),
        in_specs=[
            pl.BlockSpec((NC, BR, HALF), lambda i: (0, i, 0)),
            _DEG_SPEC,
            pl.BlockSpec((1, D), lambda i: (0, 0)),
            pl.BlockSpec((D, D), lambda i: (0, 0)),
        ],
        out_specs=pl.BlockSpec((NC, BR, HALF), lambda i: (0, i, 0)),
        out_shape=jax.ShapeDtypeStruct((NC, N, HALF), jnp.float32),
    )(agg, deg, b1, W2)


def _tc_final(agg, deg, b2):
    def body(a_r, d_r, b_r, o_r):
        dis = _dis_block(d_r)
        o_r[...] = jnp.concatenate([a_r[0], a_r[1]], axis=1) * dis + b_r[...]

    return pl.pallas_call(
        body,
        grid=(GRID,),
        in_specs=[
            pl.BlockSpec((NC, BR, HALF), lambda i: (0, i, 0)),
            _DEG_SPEC,
            pl.BlockSpec((1, D), lambda i: (0, 0)),
        ],
        out_specs=pl.BlockSpec((BR, D), lambda i: (i, 0)),
        out_shape=jax.ShapeDtypeStruct((N, D), jnp.float32),
    )(agg, deg, b2)


def kernel(x, edge_index, W1, b1, W2, b2):
    ei = edge_index.astype(jnp.int32)
    src = ei[0]
    dst = ei[1]
    pad_per_tile = EPT_PAD - EDGES_PER_TILE
    srcm = jnp.pad(src.reshape(NS, EDGES_PER_TILE),
                   ((0, 0), (0, pad_per_tile))).reshape(-1)
    dstm = jnp.pad(dst.reshape(NS, EDGES_PER_TILE),
                   ((0, 0), (0, pad_per_tile)),
                   constant_values=N_PAD - 1).reshape(-1)
    src2 = jnp.concatenate([srcm, srcm + N])
    dst_deg = jnp.pad(dst.reshape(NC * NS, DEG_EDGES_PER_TILE),
                      ((0, 0), (0, DEG_EPT_PAD - DEG_EDGES_PER_TILE)),
                      constant_values=N_PAD - 1).reshape(-1)
    ones_h = jnp.ones((DEG_CHUNK, HALF), jnp.float32)
    zeros_h = jnp.zeros((ROWS_PER_TILE, HALF), jnp.float32)

    xw = _tc_matmul1(x, W1)
    deg = _sc_degree(dst_deg, ones_h, zeros_h)
    y = _tc_scale1(xw, deg)
    agg = _sc_aggregate(src2, dstm, y.reshape(NC * N, HALF), zeros_h)
    y = _tc_layer2(agg, deg, b1.reshape(1, D), W2)
    agg = _sc_aggregate(src2, dstm, y.reshape(NC * N, HALF), zeros_h)
    return _tc_final(agg, deg, b2.reshape(1, D))


# speedup vs baseline: 1.0604x; 1.0604x over previous
import functools

import jax
import jax.numpy as jnp
from jax import lax
from jax.experimental import pallas as pl
from jax.experimental.pallas import tpu as pltpu
from jax.experimental.pallas import tpu_sc as plsc

N = 10000
D = 256
HALF = 128
E = 160000

NC = 2
NS = 16
N_PAD = 10240
ROWS_PER_TILE = N_PAD // NS
EDGES_PER_TILE = E // NS
EPT_PAD = 10080
AGG_CHUNK = 56
AGG_ITERS = EPT_PAD // AGG_CHUNK
DEG_EDGES_PER_TILE = E // (NC * NS)
DEG_CHUNK = 40
DEG_ITERS = DEG_EDGES_PER_TILE // DEG_CHUNK

BR = 400
GRID = N // BR


def _mesh():
    return plsc.VectorSubcoreMesh(core_axis_name="c", subcore_axis_name="s")


def _sc_degree(dst4, ones_hbm, zeros_hbm):
    @functools.partial(
        pl.kernel,
        out_type=jax.ShapeDtypeStruct((NC, N_PAD, HALF), jnp.float32),
        mesh=_mesh(),
        scratch_types=[
            pltpu.VMEM((DEG_ITERS, DEG_CHUNK), jnp.int32),
            pltpu.VMEM((DEG_CHUNK, HALF), jnp.float32),
            pltpu.VMEM_SHARED((N_PAD, HALF), jnp.float32),
            pltpu.SemaphoreType.DMA,
        ],
    )
    def k(dst_r, ones_r, zeros_r, d_r, didx, ones_v, acc, sem):
        c = lax.axis_index("c")
        s = lax.axis_index("s")
        row0 = s * ROWS_PER_TILE
        pltpu.sync_copy(zeros_r, acc.at[pl.ds(row0, ROWS_PER_TILE)])
        pltpu.sync_copy(ones_r, ones_v)
        pltpu.sync_copy(dst_r.at[c, s], didx)
        plsc.subcore_barrier()

        def body(i, carry):
            pltpu.async_copy(ones_v, acc.at[didx.at[i]], sem, add=True)
            return carry

        lax.fori_loop(0, DEG_ITERS, body, 0)

        def drain(i, carry):
            pltpu.make_async_copy(ones_r, ones_v, sem).wait()
            return carry

        lax.fori_loop(0, DEG_ITERS, drain, 0)
        plsc.subcore_barrier()
        sl = pl.ds(row0, ROWS_PER_TILE)
        pltpu.sync_copy(acc.at[sl], d_r.at[c, sl])

    return k(dst4, ones_hbm, zeros_hbm)


def _sl(idx_ref, i):
    return idx_ref.at[pl.ds(i * AGG_CHUNK, AGG_CHUNK)]


def _sc_aggregate(src2, dst2, y, zeros_hbm):
    @functools.partial(
        pl.kernel,
        out_type=jax.ShapeDtypeStruct((NC, N_PAD, HALF), jnp.float32),
        mesh=_mesh(),
        scratch_types=[
            pltpu.VMEM((NS * AGG_ITERS * AGG_CHUNK // NS,), jnp.int32),
            pltpu.VMEM((NS * AGG_ITERS * AGG_CHUNK // NS,), jnp.int32),
            pltpu.VMEM((AGG_CHUNK, HALF), jnp.float32),
            pltpu.VMEM((AGG_CHUNK, HALF), jnp.float32),
            pltpu.VMEM((AGG_CHUNK, HALF), jnp.float32),
            pltpu.VMEM((AGG_CHUNK, HALF), jnp.float32),
            pltpu.VMEM_SHARED((N_PAD, HALF), jnp.float32),
            pltpu.SemaphoreType.DMA,
            pltpu.SemaphoreType.DMA,
            pltpu.SemaphoreType.DMA,
            pltpu.SemaphoreType.DMA,
        ],
    )
    def k(src_r, dst_r, y_r, z_r, o_r, sidx, didx, rows0, rows1, rows2, rows3,
          acc, sem0, sem1, sem2, sem3):
        c = lax.axis_index("c")
        s = lax.axis_index("s")
        row0 = s * ROWS_PER_TILE
        pltpu.sync_copy(src_r.at[pl.ds((c * NS + s) * EPT_PAD, EPT_PAD)], sidx)
        pltpu.sync_copy(dst_r.at[pl.ds(s * EPT_PAD, EPT_PAD)], didx)
        pltpu.sync_copy(z_r, acc.at[pl.ds(row0, ROWS_PER_TILE)])
        pltpu.async_copy(y_r.at[_sl(sidx, 0)], rows0, sem0)
        pltpu.async_copy(y_r.at[_sl(sidx, 1)], rows1, sem1)
        pltpu.async_copy(y_r.at[_sl(sidx, 2)], rows2, sem2)
        plsc.subcore_barrier()

        def wait(rows, sem):
            pltpu.make_async_copy(y_r.at[_sl(sidx, 0)], rows, sem).wait()

        def body(i, carry):
            i0 = 4 * i
            wait(rows0, sem0)
            pltpu.async_copy(y_r.at[_sl(sidx, i0 + 3)], rows3, sem3)
            pltpu.sync_copy(rows0, acc.at[_sl(didx, i0)], add=True)
            wait(rows1, sem1)
            pltpu.async_copy(y_r.at[_sl(sidx, i0 + 4)], rows0, sem0)
            pltpu.sync_copy(rows1, acc.at[_sl(didx, i0 + 1)], add=True)
            wait(rows2, sem2)
            pltpu.async_copy(y_r.at[_sl(sidx, i0 + 5)], rows1, sem1)
            pltpu.sync_copy(rows2, acc.at[_sl(didx, i0 + 2)], add=True)
            wait(rows3, sem3)
            pltpu.async_copy(y_r.at[_sl(sidx, i0 + 6)], rows2, sem2)
            pltpu.sync_copy(rows3, acc.at[_sl(didx, i0 + 3)], add=True)
            return carry

        lax.fori_loop(0, (AGG_ITERS - 4) // 4, body, 0)
        wait(rows0, sem0)
        pltpu.async_copy(y_r.at[_sl(sidx, AGG_ITERS - 1)], rows3, sem3)
        pltpu.sync_copy(rows0, acc.at[_sl(didx, AGG_ITERS - 4)], add=True)
        wait(rows1, sem1)
        pltpu.sync_copy(rows1, acc.at[_sl(didx, AGG_ITERS - 3)], add=True)
        wait(rows2, sem2)
        pltpu.sync_copy(rows2, acc.at[_sl(didx, AGG_ITERS - 2)], add=True)
        wait(rows3, sem3)
        pltpu.sync_copy(rows3, acc.at[_sl(didx, AGG_ITERS - 1)], add=True)
        plsc.subcore_barrier()
        sl = pl.ds(row0, ROWS_PER_TILE)
        pltpu.sync_copy(acc.at[sl], o_r.at[c, sl])

    return k(src2, dst2, y, zeros_hbm)


def _dis_block(d_r):
    d = d_r[0, :, 0:1] + d_r[1, :, 0:1]
    return jnp.where(d > 0, lax.rsqrt(d), 0.0)


_DEG_SPEC = pl.BlockSpec((NC, BR, HALF), lambda i: (0, i, 0))


def _tc_matmul1(x, W1):
    def body(x_r, w_r, y_r):
        y_r[...] = jnp.dot(x_r[...], w_r[...],
                           preferred_element_type=jnp.float32)

    return pl.pallas_call(
        body,
        grid=(GRID,),
        in_specs=[
            pl.BlockSpec((BR, D), lambda i: (i, 0)),
            pl.BlockSpec((D, D), lambda i: (0, 0)),
        ],
        out_specs=pl.BlockSpec((BR, D), lambda i: (i, 0)),
        out_shape=jax.ShapeDtypeStruct((N, D), jnp.float32),
    )(x, W1)


def _tc_scale1(xw, deg):
    def body(xw_r, d_r, y_r):
        dis = _dis_block(d_r)
        y = xw_r[...] * dis
        y_r[0] = y[:, :HALF]
        y_r[1] = y[:, HALF:]

    return pl.pallas_call(
        body,
        grid=(GRID,),
        in_specs=[
            pl.BlockSpec((BR, D), lambda i: (i, 0)),
            _DEG_SPEC,
        ],
        out_specs=pl.BlockSpec((NC, BR, HALF), lambda i: (0, i, 0)),
        out_shape=jax.ShapeDtypeStruct((NC, N, HALF), jnp.float32),
    )(xw, deg)


def _tc_layer2(agg, deg, b1, W2):
    def body(a_r, d_r, b_r, w_r, y_r):
        dis = _dis_block(d_r)
        ag = jnp.concatenate([a_r[0], a_r[1]], axis=1)
        h = jnp.maximum(ag * dis + b_r[...], 0.0)
        y = jnp.dot(h, w_r[...], preferred_element_type=jnp.float32) * dis
        y_r[0] = y[:, :HALF]
        y_r[1] = y[:, HALF:]

    return pl.pallas_call(
        body,
        grid=(GRID,),
        in_specs=[
            pl.BlockSpec((NC, BR, HALF), lambda i: (0, i, 0)),
            _DEG_SPEC,
            pl.BlockSpec((1, D), lambda i: (0, 0)),
            pl.BlockSpec((D, D), lambda i: (0, 0)),
        ],
        out_specs=pl.BlockSpec((NC, BR, HALF), lambda i: (0, i, 0)),
        out_shape=jax.ShapeDtypeStruct((NC, N, HALF), jnp.float32),
    )(agg, deg, b1, W2)


def _tc_final(agg, deg, b2):
    def body(a_r, d_r, b_r, o_r):
        dis = _dis_block(d_r)
        o_r[...] = jnp.concatenate([a_r[0], a_r[1]], axis=1) * dis + b_r[...]

    return pl.pallas_call(
        body,
        grid=(GRID,),
        in_specs=[
            pl.BlockSpec((NC, BR, HALF), lambda i: (0, i, 0)),
            _DEG_SPEC,
            pl.BlockSpec((1, D), lambda i: (0, 0)),
        ],
        out_specs=pl.BlockSpec((BR, D), lambda i: (i, 0)),
        out_shape=jax.ShapeDtypeStruct((N, D), jnp.float32),
    )(agg, deg, b2)


def kernel(x, edge_index, W1, b1, W2, b2):
    ei = edge_index.astype(jnp.int32)
    src = ei[0]
    dst = ei[1]
    pad_per_tile = EPT_PAD - EDGES_PER_TILE
    srcm = jnp.pad(src.reshape(NS, EDGES_PER_TILE),
                   ((0, 0), (0, pad_per_tile))).reshape(-1)
    dstm = jnp.pad(dst.reshape(NS, EDGES_PER_TILE),
                   ((0, 0), (0, pad_per_tile)),
                   constant_values=N_PAD - 1).reshape(-1)
    src2 = jnp.concatenate([srcm, srcm + N])
    dst4 = dst.reshape(NC, NS, DEG_ITERS, DEG_CHUNK)
    ones_h = jnp.ones((DEG_CHUNK, HALF), jnp.float32)
    zeros_h = jnp.zeros((ROWS_PER_TILE, HALF), jnp.float32)

    xw = _tc_matmul1(x, W1)
    deg = _sc_degree(dst4, ones_h, zeros_h)
    y = _tc_scale1(xw, deg)
    agg = _sc_aggregate(src2, dstm, y.reshape(NC * N, HALF), zeros_h)
    y = _tc_layer2(agg, deg, b1.reshape(1, D), W2)
    agg = _sc_aggregate(src2, dstm, y.reshape(NC * N, HALF), zeros_h)
    return _tc_final(agg, deg, b2.reshape(1, D))


# speedup vs baseline: 1.0613x; 1.0009x over previous
import functools

import jax
import jax.numpy as jnp
from jax import lax
from jax.experimental import pallas as pl
from jax.experimental.pallas import tpu as pltpu
from jax.experimental.pallas import tpu_sc as plsc

N = 10000
D = 256
HALF = 128
E = 160000

NC = 2
NS = 16
N_PAD = 10240
ROWS_PER_TILE = N_PAD // NS
EDGES_PER_TILE = E // NS
EPT_PAD = 10080
AGG_CHUNK = 56
AGG_ITERS = EPT_PAD // AGG_CHUNK
DEG_EDGES_PER_TILE = E // (NC * NS)
DEG_CHUNK = 40
DEG_ITERS = DEG_EDGES_PER_TILE // DEG_CHUNK

BR = 400
GRID = N // BR


def _mesh():
    return plsc.VectorSubcoreMesh(core_axis_name="c", subcore_axis_name="s")


def _sc_degree(dst4, ones_hbm, zeros_hbm):
    @functools.partial(
        pl.kernel,
        out_type=jax.ShapeDtypeStruct((NC, N_PAD, HALF), jnp.float32),
        mesh=_mesh(),
        scratch_types=[
            pltpu.VMEM((DEG_ITERS, DEG_CHUNK), jnp.int32),
            pltpu.VMEM((DEG_CHUNK, HALF), jnp.float32),
            pltpu.VMEM_SHARED((N_PAD, HALF), jnp.float32),
            pltpu.SemaphoreType.DMA,
        ],
    )
    def k(dst_r, ones_r, zeros_r, d_r, didx, ones_v, acc, sem):
        c = lax.axis_index("c")
        s = lax.axis_index("s")
        row0 = s * ROWS_PER_TILE
        pltpu.sync_copy(zeros_r, acc.at[pl.ds(row0, ROWS_PER_TILE)])
        pltpu.sync_copy(ones_r, ones_v)
        pltpu.sync_copy(dst_r.at[c, s], didx)
        plsc.subcore_barrier()

        def body(i, carry):
            pltpu.async_copy(ones_v, acc.at[didx.at[i]], sem, add=True)
            return carry

        lax.fori_loop(0, DEG_ITERS, body, 0)

        def drain(i, carry):
            pltpu.make_async_copy(ones_r, ones_v, sem).wait()
            return carry

        lax.fori_loop(0, DEG_ITERS, drain, 0)
        plsc.subcore_barrier()
        sl = pl.ds(row0, ROWS_PER_TILE)
        pltpu.sync_copy(acc.at[sl], d_r.at[c, sl])

    return k(dst4, ones_hbm, zeros_hbm)


def _sl(idx_ref, i):
    return idx_ref.at[pl.ds(i * AGG_CHUNK, AGG_CHUNK)]


def _sc_aggregate(src2, dst2, y, zeros_hbm):
    @functools.partial(
        pl.kernel,
        out_type=jax.ShapeDtypeStruct((NC, N_PAD, HALF), jnp.float32),
        mesh=_mesh(),
        scratch_types=[
            pltpu.VMEM((EPT_PAD,), jnp.int32),
            pltpu.VMEM((EPT_PAD,), jnp.int32),
            pltpu.VMEM((AGG_CHUNK, HALF), jnp.float32),
            pltpu.VMEM((AGG_CHUNK, HALF), jnp.float32),
            pltpu.VMEM((AGG_CHUNK, HALF), jnp.float32),
            pltpu.VMEM((AGG_CHUNK, HALF), jnp.float32),
            pltpu.VMEM_SHARED((N_PAD, HALF), jnp.float32),
            pltpu.SemaphoreType.DMA,
            pltpu.SemaphoreType.DMA,
            pltpu.SemaphoreType.DMA,
            pltpu.SemaphoreType.DMA,
        ],
    )
    def k(src_r, dst_r, y_r, z_r, o_r, sidx, didx, rows0, rows1, rows2, rows3,
          acc, sem0, sem1, sem2, sem3):
        c = lax.axis_index("c")
        s = lax.axis_index("s")
        row0 = s * ROWS_PER_TILE
        pltpu.sync_copy(src_r.at[pl.ds((c * NS + s) * EPT_PAD, EPT_PAD)], sidx)
        pltpu.sync_copy(dst_r.at[pl.ds(s * EPT_PAD, EPT_PAD)], didx)
        pltpu.sync_copy(z_r, acc.at[pl.ds(row0, ROWS_PER_TILE)])
        pltpu.async_copy(y_r.at[_sl(sidx, 0)], rows0, sem0)
        pltpu.async_copy(y_r.at[_sl(sidx, 1)], rows1, sem1)
        pltpu.async_copy(y_r.at[_sl(sidx, 2)], rows2, sem2)
        plsc.subcore_barrier()

        def wait(rows, sem):
            pltpu.make_async_copy(y_r.at[_sl(sidx, 0)], rows, sem).wait()

        def body(i, carry):
            i0 = 4 * i
            wait(rows0, sem0)
            pltpu.async_copy(y_r.at[_sl(sidx, i0 + 3)], rows3, sem3)
            pltpu.sync_copy(rows0, acc.at[_sl(didx, i0)], add=True)
            wait(rows1, sem1)
            pltpu.async_copy(y_r.at[_sl(sidx, i0 + 4)], rows0, sem0)
            pltpu.sync_copy(rows1, acc.at[_sl(didx, i0 + 1)], add=True)
            wait(rows2, sem2)
            pltpu.async_copy(y_r.at[_sl(sidx, i0 + 5)], rows1, sem1)
            pltpu.sync_copy(rows2, acc.at[_sl(didx, i0 + 2)], add=True)
            wait(rows3, sem3)
            pltpu.async_copy(y_r.at[_sl(sidx, i0 + 6)], rows2, sem2)
            pltpu.sync_copy(rows3, acc.at[_sl(didx, i0 + 3)], add=True)
            return carry

        lax.fori_loop(0, (AGG_ITERS - 4) // 4, body, 0)
        wait(rows0, sem0)
        pltpu.async_copy(y_r.at[_sl(sidx, AGG_ITERS - 1)], rows3, sem3)
        pltpu.sync_copy(rows0, acc.at[_sl(didx, AGG_ITERS - 4)], add=True)
        wait(rows1, sem1)
        pltpu.sync_copy(rows1, acc.at[_sl(didx, AGG_ITERS - 3)], add=True)
        wait(rows2, sem2)
        pltpu.sync_copy(rows2, acc.at[_sl(didx, AGG_ITERS - 2)], add=True)
        wait(rows3, sem3)
        pltpu.sync_copy(rows3, acc.at[_sl(didx, AGG_ITERS - 1)], add=True)
        plsc.subcore_barrier()
        sl = pl.ds(row0, ROWS_PER_TILE)
        pltpu.sync_copy(acc.at[sl], o_r.at[c, sl])

    return k(src2, dst2, y, zeros_hbm)


def _dis_block(d_r):
    d = d_r[0, :, 0:1] + d_r[1, :, 0:1]
    return jnp.where(d > 0, lax.rsqrt(d), 0.0)


_DEG_SPEC = pl.BlockSpec((NC, BR, HALF), lambda i: (0, i, 0))


def _tc_matmul1(x, W1):
    def body(x_r, w_r, y_r):
        y_r[...] = jnp.dot(x_r[...], w_r[...],
                           preferred_element_type=jnp.float32)

    return pl.pallas_call(
        body,
        grid=(GRID,),
        in_specs=[
            pl.BlockSpec((BR, D), lambda i: (i, 0)),
            pl.BlockSpec((D, D), lambda i: (0, 0)),
        ],
        out_specs=pl.BlockSpec((BR, D), lambda i: (i, 0)),
        out_shape=jax.ShapeDtypeStruct((N, D), jnp.float32),
    )(x, W1)


def _tc_scale1(xw, deg):
    def body(xw_r, d_r, y_r):
        dis = _dis_block(d_r)
        y = xw_r[...] * dis
        y_r[0] = y[:, :HALF]
        y_r[1] = y[:, HALF:]

    return pl.pallas_call(
        body,
        grid=(GRID,),
        in_specs=[
            pl.BlockSpec((BR, D), lambda i: (i, 0)),
            _DEG_SPEC,
        ],
        out_specs=pl.BlockSpec((NC, BR, HALF), lambda i: (0, i, 0)),
        out_shape=jax.ShapeDtypeStruct((NC, N, HALF), jnp.float32),
    )(xw, deg)


def _tc_layer2(agg, deg, b1, W2):
    def body(a_r, d_r, b_r, w_r, y_r):
        dis = _dis_block(d_r)
        ag = jnp.concatenate([a_r[0], a_r[1]], axis=1)
        h = jnp.maximum(ag * dis + b_r[...], 0.0)
        y = jnp.dot(h, w_r[...], preferred_element_type=jnp.float32) * dis
        y_r[0] = y[:, :HALF]
        y_r[1] = y[:, HALF:]

    return pl.pallas_call(
        body,
        grid=(GRID,),
        in_specs=[
            pl.BlockSpec((NC, BR, HALF), lambda i: (0, i, 0)),
            _DEG_SPEC,
            pl.BlockSpec((1, D), lambda i: (0, 0)),
            pl.BlockSpec((D, D), lambda i: (0, 0)),
        ],
        out_specs=pl.BlockSpec((NC, BR, HALF), lambda i: (0, i, 0)),
        out_shape=jax.ShapeDtypeStruct((NC, N, HALF), jnp.float32),
    )(agg, deg, b1, W2)


def _tc_final(agg, deg, b2):
    def body(a_r, d_r, b_r, o_r):
        dis = _dis_block(d_r)
        o_r[...] = jnp.concatenate([a_r[0], a_r[1]], axis=1) * dis + b_r[...]

    return pl.pallas_call(
        body,
        grid=(GRID,),
        in_specs=[
            pl.BlockSpec((NC, BR, HALF), lambda i: (0, i, 0)),
            _DEG_SPEC,
            pl.BlockSpec((1, D), lambda i: (0, 0)),
        ],
        out_specs=pl.BlockSpec((BR, D), lambda i: (i, 0)),
        out_shape=jax.ShapeDtypeStruct((N, D), jnp.float32),
    )(agg, deg, b2)


def kernel(x, edge_index, W1, b1, W2, b2):
    ei = edge_index.astype(jnp.int32)
    src = ei[0]
    dst = ei[1]
    pad_per_tile = EPT_PAD - EDGES_PER_TILE
    srcm = jnp.pad(src.reshape(NS, EDGES_PER_TILE),
                   ((0, 0), (0, pad_per_tile))).reshape(-1)
    dstm = jnp.pad(dst.reshape(NS, EDGES_PER_TILE),
                   ((0, 0), (0, pad_per_tile)),
                   constant_values=N_PAD - 1).reshape(-1)
    src2 = jnp.concatenate([srcm, srcm + N])
    dst4 = dst.reshape(NC, NS, DEG_ITERS, DEG_CHUNK)
    ones_h = jnp.ones((DEG_CHUNK, HALF), jnp.float32)
    zeros_h = jnp.zeros((ROWS_PER_TILE, HALF), jnp.float32)

    xw = _tc_matmul1(x, W1)
    deg = _sc_degree(dst4, ones_h, zeros_h)
    y = _tc_scale1(xw, deg)
    agg = _sc_aggregate(src2, dstm, y.reshape(NC * N, HALF), zeros_h)
    y = _tc_layer2(agg, deg, b1.reshape(1, D), W2)
    agg = _sc_aggregate(src2, dstm, y.reshape(NC * N, HALF), zeros_h)
    return _tc_final(agg, deg, b2.reshape(1, D))
